# baseline (speedup 1.0000x reference)
import functools

import jax
import jax.numpy as jnp
from jax import lax
from jax.experimental import pallas as pl
from jax.experimental.pallas import tpu as pltpu
from jax.experimental.pallas import tpu_sc as plsc

N_ROWS = 1000000
N_COLS = 100000
RANK_K = 32
BATCH = 16384

NC = 2
NS = 16
NW = NC * NS
BPW = BATCH // NW
CH = 128
NCH = BPW // CH
LANES = 16
NG = BPW // LANES


def _lfm_body(i_hbm, j_hbm, ut_hbm, vt_hbm, out_hbm,
              idx_i, idx_j, u_vals, v_vals, out_loc, sem):
  wid = lax.axis_index("s") * NC + lax.axis_index("c")
  base = wid * BPW

  with jax.named_scope("idx_stage"):
    pltpu.sync_copy(i_hbm.at[pl.ds(base, BPW)], idx_i)
    pltpu.sync_copy(j_hbm.at[pl.ds(base, BPW)], idx_j)

  with jax.named_scope("gather"):
    copies = []
    for k in range(RANK_K):
      for c in range(NCH):
        sl = pl.ds(c * CH, CH)
        copies.append(pltpu.async_copy(
            ut_hbm.at[k].at[idx_i.at[sl]], u_vals.at[k, sl], sem))
        copies.append(pltpu.async_copy(
            vt_hbm.at[k].at[idx_j.at[sl]], v_vals.at[k, sl], sem))
    for cp in copies:
      cp.wait()

  with jax.named_scope("dot"):
    def group(g, carry):
      sl = pl.ds(g * LANES, LANES)
      acc = u_vals[0, sl] * v_vals[0, sl]
      for k in range(1, RANK_K):
        acc = acc + u_vals[k, sl] * v_vals[k, sl]
      out_loc[sl] = acc
      return carry

    lax.fori_loop(0, NG, group, 0)

  with jax.named_scope("out_copy"):
    pltpu.sync_copy(out_loc, out_hbm.at[pl.ds(base, BPW)])


@functools.partial(
    pl.kernel,
    out_type=jax.ShapeDtypeStruct((BATCH,), jnp.float32),
    mesh=plsc.VectorSubcoreMesh(core_axis_name="c", subcore_axis_name="s"),
    compiler_params=pltpu.CompilerParams(
        needs_layout_passes=False, use_tc_tiling_on_sc=False),
    scratch_types=[
        pltpu.VMEM((BPW,), jnp.int32),
        pltpu.VMEM((BPW,), jnp.int32),
        pltpu.VMEM((RANK_K, BPW), jnp.float32),
        pltpu.VMEM((RANK_K, BPW), jnp.float32),
        pltpu.VMEM((BPW,), jnp.float32),
        pltpu.SemaphoreType.DMA,
    ],
)
def _lfm_kernel(*refs):
  _lfm_body(*refs)


def kernel(i, j, U, V):
  return _lfm_kernel(i, j, U.T, V.T)


# speedup vs baseline: 4.7105x; 4.7105x over previous
import functools

import jax
import jax.numpy as jnp
from jax import lax
from jax.experimental import pallas as pl
from jax.experimental.pallas import tpu as pltpu
from jax.experimental.pallas import tpu_sc as plsc

N_ROWS = 1000000
N_COLS = 100000
RANK_K = 32
BATCH = 16384

PACK = 4
UB = N_ROWS // PACK
VB = N_COLS // PACK
W = PACK * RANK_K

NC = 2
NS = 16
NW = NC * NS
BPW = BATCH // NW
CH = 128
NCH = BPW // CH
LANES = 16
GPC = CH // LANES


def _lfm_body(i_hbm, j_hbm, u4_hbm, v4_hbm, out_hbm,
              idx_i, idx_j, qi, qj, u_wide, v_wide, out_loc, sem):
  wid = lax.axis_index("s") * NC + lax.axis_index("c")
  base = wid * BPW

  with jax.named_scope("idx_stage"):
    pltpu.sync_copy(i_hbm.at[pl.ds(base, BPW)], idx_i)
    pltpu.sync_copy(j_hbm.at[pl.ds(base, BPW)], idx_j)
    for r in range(BPW // LANES):
      sl = pl.ds(r * LANES, LANES)
      qi[sl] = lax.shift_right_logical(idx_i[sl], 2)
      qj[sl] = lax.shift_right_logical(idx_j[sl], 2)

  def fire(c):
    sl = pl.ds(c * CH, CH)
    p = c % 2
    return (pltpu.async_copy(u4_hbm.at[qi.at[sl]], u_wide.at[p], sem),
            pltpu.async_copy(v4_hbm.at[qj.at[sl]], v_wide.at[p], sem))

  lane = lax.iota(jnp.int32, LANES)

  def compute(c):
    p = c % 2
    pv = jnp.full((LANES,), p, jnp.int32)
    for g in range(GPC):
      sl = pl.ds(c * CH + g * LANES, LANES)
      iv = idx_i[sl]
      jv = idx_j[sl]
      rowv = g * LANES + lane
      cu = (iv & 3) * RANK_K
      cv = (jv & 3) * RANK_K
      acc = jnp.zeros((LANES,), jnp.float32)
      for k in range(RANK_K):
        ug = plsc.load_gather(u_wide, [pv, rowv, cu + k])
        vg = plsc.load_gather(v_wide, [pv, rowv, cv + k])
        acc = acc + ug * vg
      out_loc[sl] = acc

  with jax.named_scope("gather_dot"):
    inflight = fire(0)
    for c in range(NCH):
      for cp in inflight:
        cp.wait()
      if c + 1 < NCH:
        inflight = fire(c + 1)
      compute(c)

  with jax.named_scope("out_copy"):
    pltpu.sync_copy(out_loc, out_hbm.at[pl.ds(base, BPW)])


@functools.partial(
    pl.kernel,
    out_type=jax.ShapeDtypeStruct((BATCH,), jnp.float32),
    mesh=plsc.VectorSubcoreMesh(core_axis_name="c", subcore_axis_name="s"),
    compiler_params=pltpu.CompilerParams(
        needs_layout_passes=False, use_tc_tiling_on_sc=True),
    scratch_types=[
        pltpu.VMEM((BPW,), jnp.int32),
        pltpu.VMEM((BPW,), jnp.int32),
        pltpu.VMEM((BPW,), jnp.int32),
        pltpu.VMEM((BPW,), jnp.int32),
        pltpu.VMEM((2, CH, W), jnp.float32),
        pltpu.VMEM((2, CH, W), jnp.float32),
        pltpu.VMEM((BPW,), jnp.float32),
        pltpu.SemaphoreType.DMA,
    ],
)
def _lfm_kernel(*refs):
  _lfm_body(*refs)


def kernel(i, j, U, V):
  return _lfm_kernel(i, j, U.reshape(UB, W), V.reshape(VB, W))


# speedup vs baseline: 12.1174x; 2.5724x over previous
import functools

import jax
import jax.numpy as jnp
from jax import lax
from jax.experimental import pallas as pl
from jax.experimental.pallas import tpu as pltpu
from jax.experimental.pallas import tpu_sc as plsc

N_ROWS = 1000000
N_COLS = 100000
RANK_K = 32
BATCH = 16384

PACK = 4
VB = N_COLS // PACK
W = PACK * RANK_K

NC = 2
NS = 16
NW = NC * NS
BPW = BATCH // NW
CH = 128
NCH = BPW // CH
LANES = 16
GPC = CH // LANES
RING = 16
NWAVE = CH // RING


def _lfm_body(i_hbm, j_hbm, ut_hbm, v4_hbm, out_hbm,
              idx_i, idx_j, qj, u_tiles, v_wide, u_rows, out_loc,
              sem, usem):
  wid = lax.axis_index("s") * NC + lax.axis_index("c")
  base = wid * BPW
  lane = lax.iota(jnp.int32, LANES)

  with jax.named_scope("idx_stage"):
    pltpu.sync_copy(i_hbm.at[pl.ds(base, BPW)], idx_i)
    pltpu.sync_copy(j_hbm.at[pl.ds(base, BPW)], idx_j)
    for r in range(BPW // LANES):
      sl = pl.ds(r * LANES, LANES)
      qj[sl] = lax.shift_right_logical(idx_j[sl], 2)

  def fire_v(c):
    sl = pl.ds(c * CH, CH)
    return pltpu.async_copy(v4_hbm.at[qj.at[sl]], v_wide.at[c % 2], sem)

  def iget(r):
    v = idx_i[pl.ds((r // LANES) * LANES, LANES)]
    return jnp.sum(jnp.where(lane == r % LANES, v, 0))

  k0 = lane
  k1 = lane + LANES

  vfly = fire_v(0)

  for c in range(NCH):
    with jax.named_scope("u_tile_loop"):
      def wave(w, carry, c=c):
        r0 = c * CH + w * RING
        for s in range(RING):
          n = iget(r0 + s)
          t = pl.multiple_of((n // 128) * 128, 128)
          pltpu.async_copy(ut_hbm.at[:, pl.ds(t, 128)], u_tiles.at[s],
                           usem.at[s])
        for s in range(RING):
          pltpu.make_async_copy(ut_hbm.at[:, pl.ds(0, 128)], u_tiles.at[s],
                                usem.at[s]).wait()
          n = iget(r0 + s)
          col = jnp.zeros((LANES,), jnp.int32) + n % 128
          sv = jnp.zeros((LANES,), jnp.int32) + s
          u0 = plsc.load_gather(u_tiles, [sv, k0, col])
          u1 = plsc.load_gather(u_tiles, [sv, k1, col])
          rr = w * RING + s
          u_rows[rr, pl.ds(0, LANES)] = u0
          u_rows[rr, pl.ds(LANES, LANES)] = u1
        return carry

      lax.fori_loop(0, NWAVE, wave, 0)

    with jax.named_scope("dot"):
      vfly.wait()
      if c + 1 < NCH:
        vfly = fire_v(c + 1)
      pv = jnp.zeros((LANES,), jnp.int32) + (c % 2)

      def group(g, carry, c=c, pv=pv):
        sl = pl.ds(c * CH + g * LANES, LANES)
        jv = idx_j[sl]
        rowv = g * LANES + lane
        cv = (jv & 3) * RANK_K
        zero = jnp.zeros((LANES,), jnp.int32)
        acc = jnp.zeros((LANES,), jnp.float32)
        for k in range(RANK_K):
          ug = plsc.load_gather(u_rows, [rowv, zero + k])
          vg = plsc.load_gather(v_wide, [pv, rowv, cv + k])
          acc = acc + ug * vg
        out_loc[sl] = acc
        return carry

      lax.fori_loop(0, GPC, group, 0)

  with jax.named_scope("out_copy"):
    pltpu.sync_copy(out_loc, out_hbm.at[pl.ds(base, BPW)])


@functools.partial(
    pl.kernel,
    out_type=jax.ShapeDtypeStruct((BATCH,), jnp.float32),
    mesh=plsc.VectorSubcoreMesh(core_axis_name="c", subcore_axis_name="s"),
    compiler_params=pltpu.CompilerParams(
        needs_layout_passes=False, use_tc_tiling_on_sc=True),
    scratch_types=[
        pltpu.VMEM((BPW,), jnp.int32),
        pltpu.VMEM((BPW,), jnp.int32),
        pltpu.VMEM((BPW,), jnp.int32),
        pltpu.VMEM((RING, RANK_K, 128), jnp.float32),
        pltpu.VMEM((2, CH, W), jnp.float32),
        pltpu.VMEM((CH, RANK_K), jnp.float32),
        pltpu.VMEM((BPW,), jnp.float32),
        pltpu.SemaphoreType.DMA,
        pltpu.SemaphoreType.DMA((RING,)),
    ],
)
def _lfm_kernel(*refs):
  _lfm_body(*refs)


def kernel(i, j, U, V):
  return _lfm_kernel(i, j, U.T, V.reshape(VB, W))


# speedup vs baseline: 13.3613x; 1.1027x over previous
import functools

import jax
import jax.numpy as jnp
from jax import lax
from jax.experimental import pallas as pl
from jax.experimental.pallas import tpu as pltpu
from jax.experimental.pallas import tpu_sc as plsc

N_ROWS = 1000000
N_COLS = 100000
RANK_K = 32
BATCH = 16384

PACK = 4
VB = N_COLS // PACK
W = PACK * RANK_K

NC = 2
NS = 16
NW = NC * NS
BPW = BATCH // NW
CH = 128
NCH = BPW // CH
LANES = 16
GPC = CH // LANES
RING = 16
NWAVE = CH // RING


def _lfm_body(i_hbm, j_hbm, ut_hbm, v4_hbm, out_hbm,
              idx_i, idx_j, qj, colbuf, u_tiles, v_wide, u_rows, out_loc,
              sem, usem):
  wid = lax.axis_index("s") * NC + lax.axis_index("c")
  base = wid * BPW
  lane = lax.iota(jnp.int32, LANES)
  zero = jnp.zeros((LANES,), jnp.int32)
  k0 = lane
  k1 = lane + LANES

  with jax.named_scope("idx_stage"):
    pltpu.sync_copy(i_hbm.at[pl.ds(base, BPW)], idx_i)
    pltpu.sync_copy(j_hbm.at[pl.ds(base, BPW)], idx_j)
    for r in range(BPW // LANES):
      sl = pl.ds(r * LANES, LANES)
      qj[sl] = lax.shift_right_logical(idx_j[sl], 2)
      colbuf[sl] = idx_i[sl] & 127

  def fire_v(c):
    sl = pl.ds(c * CH, CH)
    return pltpu.async_copy(v4_hbm.at[qj.at[sl]], v_wide.at[c % 2], sem)

  def fire_slot(vwave, s):
    n = jnp.sum(jnp.where(lane == s, vwave, 0))
    t = pl.multiple_of((n // 128) * 128, 128)
    pltpu.async_copy(ut_hbm.at[:, pl.ds(t, 128)], u_tiles.at[s],
                     usem.at[s])

  def fire_wave(c, wv):
    vwave = idx_i[pl.ds(c * CH + wv * LANES, LANES)]
    for s in range(RING):
      fire_slot(vwave, s)

  def extract_slot(c, wv, s):
    rr = wv * LANES + s
    colv = plsc.load_gather(colbuf, [zero + (c * CH + rr)])
    sv = jnp.zeros((LANES,), jnp.int32) + s
    u0 = plsc.load_gather(u_tiles, [sv, k0, colv])
    u1 = plsc.load_gather(u_tiles, [sv, k1, colv])
    u_rows[rr, pl.ds(0, LANES)] = u0
    u_rows[rr, pl.ds(LANES, LANES)] = u1

  def wait_slot(s):
    pltpu.make_async_copy(ut_hbm.at[:, pl.ds(0, 128)], u_tiles.at[s],
                          usem.at[s]).wait()

  vfly = fire_v(0)

  for c in range(NCH):
    with jax.named_scope("u_tile_loop"):
      fire_wave(c, 0)

      def wavebody(wv, carry, c=c):
        vwave = idx_i[pl.ds(c * CH + wv * LANES, LANES)]
        for s in range(RING):
          wait_slot(s)
          extract_slot(c, wv - 1, s)
          fire_slot(vwave, s)
        return carry

      lax.fori_loop(1, NWAVE, wavebody, 0)
      for s in range(RING):
        wait_slot(s)
        extract_slot(c, NWAVE - 1, s)

    with jax.named_scope("dot"):
      vfly.wait()
      if c + 1 < NCH:
        vfly = fire_v(c + 1)
      pv = jnp.zeros((LANES,), jnp.int32) + (c % 2)

      def group(g, carry, c=c, pv=pv):
        sl = pl.ds(c * CH + g * LANES, LANES)
        jv = idx_j[sl]
        rowv = g * LANES + lane
        cv = (jv & 3) * RANK_K
        acc = jnp.zeros((LANES,), jnp.float32)
        for k in range(RANK_K):
          ug = plsc.load_gather(u_rows, [rowv, zero + k])
          vg = plsc.load_gather(v_wide, [pv, rowv, cv + k])
          acc = acc + ug * vg
        out_loc[sl] = acc
        return carry

      lax.fori_loop(0, GPC, group, 0)

  with jax.named_scope("out_copy"):
    pltpu.sync_copy(out_loc, out_hbm.at[pl.ds(base, BPW)])


@functools.partial(
    pl.kernel,
    out_type=jax.ShapeDtypeStruct((BATCH,), jnp.float32),
    mesh=plsc.VectorSubcoreMesh(core_axis_name="c", subcore_axis_name="s"),
    compiler_params=pltpu.CompilerParams(
        needs_layout_passes=False, use_tc_tiling_on_sc=True),
    scratch_types=[
        pltpu.VMEM((BPW,), jnp.int32),
        pltpu.VMEM((BPW,), jnp.int32),
        pltpu.VMEM((BPW,), jnp.int32),
        pltpu.VMEM((BPW,), jnp.int32),
        pltpu.VMEM((RING, RANK_K, 128), jnp.float32),
        pltpu.VMEM((2, CH, W), jnp.float32),
        pltpu.VMEM((CH, RANK_K), jnp.float32),
        pltpu.VMEM((BPW,), jnp.float32),
        pltpu.SemaphoreType.DMA,
        pltpu.SemaphoreType.DMA((RING,)),
    ],
)
def _lfm_kernel(*refs):
  _lfm_body(*refs)


def kernel(i, j, U, V):
  return _lfm_kernel(i, j, U.T, V.reshape(VB, W))


# speedup vs baseline: 13.9139x; 1.0414x over previous
import functools

import jax
import jax.numpy as jnp
from jax import lax
from jax.experimental import pallas as pl
from jax.experimental.pallas import tpu as pltpu
from jax.experimental.pallas import tpu_sc as plsc

N_ROWS = 1000000
N_COLS = 100000
RANK_K = 32
BATCH = 16384

PACK = 4
VB = N_COLS // PACK
W = PACK * RANK_K

NC = 2
NS = 16
NW = NC * NS
BPW = BATCH // NW
CH = 128
NCH = BPW // CH
LANES = 16
GPC = CH // LANES
RING = 16
NWAVE = CH // RING


def _lfm_body(i_hbm, j_hbm, ut_hbm, v4_hbm, out_hbm,
              idx_i, idx_j, qj, colbuf, u_tiles, v_wide, u_rows, out_loc,
              sem, usem):
  wid = lax.axis_index("s") * NC + lax.axis_index("c")
  base = wid * BPW
  lane = lax.iota(jnp.int32, LANES)
  zero = jnp.zeros((LANES,), jnp.int32)
  k0 = lane
  k1 = lane + LANES

  with jax.named_scope("idx_stage"):
    pltpu.sync_copy(i_hbm.at[pl.ds(base, BPW)], idx_i)
    pltpu.sync_copy(j_hbm.at[pl.ds(base, BPW)], idx_j)
    for r in range(BPW // LANES):
      sl = pl.ds(r * LANES, LANES)
      qj[sl] = lax.shift_right_logical(idx_j[sl], 2)
      colbuf[sl] = idx_i[sl] & 127

  def fire_v(c):
    sl = pl.ds(c * CH, CH)
    return pltpu.async_copy(v4_hbm.at[qj.at[sl]], v_wide.at[c % 2], sem)

  def fire_slot(vwave, s):
    n = jnp.sum(jnp.where(lane == s, vwave, 0))
    t = pl.multiple_of((n // 128) * 128, 128)
    pltpu.async_copy(ut_hbm.at[:, pl.ds(t, 128)], u_tiles.at[s],
                     usem.at[s])

  def fire_wave(c, wv):
    vwave = idx_i[pl.ds(c * CH + wv * LANES, LANES)]
    for s in range(RING):
      fire_slot(vwave, s)

  def extract_slot(c, wv, s):
    rr = wv * LANES + s
    colv = plsc.load_gather(colbuf, [zero + (c * CH + rr)])
    sv = jnp.zeros((LANES,), jnp.int32) + s
    u0 = plsc.load_gather(u_tiles, [sv, k0, colv])
    u1 = plsc.load_gather(u_tiles, [sv, k1, colv])
    u_rows[rr, pl.ds(0, LANES)] = u0
    u_rows[rr, pl.ds(LANES, LANES)] = u1

  def wait_slot(s):
    pltpu.make_async_copy(ut_hbm.at[:, pl.ds(0, 128)], u_tiles.at[s],
                          usem.at[s]).wait()

  vfly = fire_v(0)

  fire_wave(0, 0)
  for c in range(NCH):
    with jax.named_scope("u_tile_loop"):
      def wavebody(wv, carry, c=c):
        vwave = idx_i[pl.ds(c * CH + wv * LANES, LANES)]
        for s in range(RING):
          wait_slot(s)
          extract_slot(c, wv - 1, s)
          fire_slot(vwave, s)
        return carry

      lax.fori_loop(1, NWAVE, wavebody, 0)
      if c + 1 < NCH:
        vnext = idx_i[pl.ds((c + 1) * CH, LANES)]
      for s in range(RING):
        wait_slot(s)
        extract_slot(c, NWAVE - 1, s)
        if c + 1 < NCH:
          fire_slot(vnext, s)

    with jax.named_scope("dot"):
      vfly.wait()
      if c + 1 < NCH:
        vfly = fire_v(c + 1)
      pv = jnp.zeros((LANES,), jnp.int32) + (c % 2)

      def group(g, carry, c=c, pv=pv):
        sl = pl.ds(c * CH + g * LANES, LANES)
        jv = idx_j[sl]
        rowv = g * LANES + lane
        cv = (jv & 3) * RANK_K
        acc = jnp.zeros((LANES,), jnp.float32)
        for k in range(RANK_K):
          ug = plsc.load_gather(u_rows, [rowv, zero + k])
          vg = plsc.load_gather(v_wide, [pv, rowv, cv + k])
          acc = acc + ug * vg
        out_loc[sl] = acc
        return carry

      lax.fori_loop(0, GPC, group, 0)

  with jax.named_scope("out_copy"):
    pltpu.sync_copy(out_loc, out_hbm.at[pl.ds(base, BPW)])


@functools.partial(
    pl.kernel,
    out_type=jax.ShapeDtypeStruct((BATCH,), jnp.float32),
    mesh=plsc.VectorSubcoreMesh(core_axis_name="c", subcore_axis_name="s"),
    compiler_params=pltpu.CompilerParams(
        needs_layout_passes=False, use_tc_tiling_on_sc=True),
    scratch_types=[
        pltpu.VMEM((BPW,), jnp.int32),
        pltpu.VMEM((BPW,), jnp.int32),
        pltpu.VMEM((BPW,), jnp.int32),
        pltpu.VMEM((BPW,), jnp.int32),
        pltpu.VMEM((RING, RANK_K, 128), jnp.float32),
        pltpu.VMEM((2, CH, W), jnp.float32),
        pltpu.VMEM((CH, RANK_K), jnp.float32),
        pltpu.VMEM((BPW,), jnp.float32),
        pltpu.SemaphoreType.DMA,
        pltpu.SemaphoreType.DMA((RING,)),
    ],
)
def _lfm_kernel(*refs):
  _lfm_body(*refs)


def kernel(i, j, U, V):
  return _lfm_kernel(i, j, U.T, V.reshape(VB, W))
